# Initial kernel scaffold; baseline (speedup 1.0000x reference)
import jax
import jax.numpy as jnp
from jax.experimental import pallas as pl


def kernel(x, edge_index, W, bias):
    raise NotImplementedError("write your pallas kernel here")


# speedup vs baseline: 2.6124x; 2.6124x over previous
import functools

import jax
import jax.numpy as jnp
from jax import lax
from jax.experimental import pallas as pl
from jax.experimental.pallas import tpu as pltpu
from jax.experimental.pallas import tpu_sc as plsc

C = 128
OUT = 128
K = 32
NW = 32
CN = 4
CE = CN * K
COLS = OUT // 16


def _tables_body(x_ref, a1_ref, a2_ref, b_ref, u_ref, v_ref):
    x = x_ref[...]
    u = lax.dot_general(x, a1_ref[...], (((0,), (1,)), ((), ())),
                        preferred_element_type=jnp.float32)
    v = lax.dot_general(x, a2_ref[...], (((0,), (1,)), ((), ())),
                        preferred_element_type=jnp.float32)
    u_ref[...] = u + b_ref[...]
    v_ref[...] = v


def _make_tables(x_pad, a1, a2, b, n_pad, bn):
    grid = (n_pad // bn,)
    return pl.pallas_call(
        _tables_body,
        grid=grid,
        in_specs=[
            pl.BlockSpec((C, bn), lambda i: (0, i)),
            pl.BlockSpec((OUT, C), lambda i: (0, 0)),
            pl.BlockSpec((OUT, C), lambda i: (0, 0)),
            pl.BlockSpec((1, OUT), lambda i: (0, 0)),
        ],
        out_specs=[
            pl.BlockSpec((bn, OUT), lambda i: (i, 0)),
            pl.BlockSpec((bn, OUT), lambda i: (i, 0)),
        ],
        out_shape=[
            jax.ShapeDtypeStruct((n_pad, OUT), jnp.float32),
            jax.ShapeDtypeStruct((n_pad, OUT), jnp.float32),
        ],
    )(x_pad, a1, a2, b)


def _sc_body(nchunk, npw,
             u_hbm, v_hbm, i1_hbm, i0_hbm, mx_hbm, mn_hbm,
             i1_v, i0_v, ur_v, vr_v, mx_v, mn_v, sem1, sem2):
    cid = lax.axis_index("c")
    sid = lax.axis_index("s")
    wid = sid * 2 + cid
    base = wid * npw

    def chunk_body(ci, carry):
        nb = base + ci * CN
        eb = nb * K
        pltpu.sync_copy(i1_hbm.at[pl.ds(eb, CE)], i1_v)
        pltpu.sync_copy(i0_hbm.at[pl.ds(eb, CE)], i0_v)
        cp1 = pltpu.async_copy(u_hbm.at[i1_v], ur_v, sem1)
        cp2 = pltpu.async_copy(v_hbm.at[i0_v], vr_v, sem2)
        cp1.wait()
        cp2.wait()
        for n in range(CN):
            def k_body(k, acc):
                row = n * K + k
                out = []
                for j in range(COLS):
                    z = (ur_v[row, pl.ds(j * 16, 16)]
                         + vr_v[row, pl.ds(j * 16, 16)])
                    out.append(jnp.maximum(acc[j], z))
                    out.append(jnp.minimum(acc[COLS + j], z))
                return tuple(out[0::2]) + tuple(out[1::2])

            init = (tuple(jnp.full((16,), -jnp.inf, jnp.float32) for _ in range(COLS))
                    + tuple(jnp.full((16,), jnp.inf, jnp.float32) for _ in range(COLS)))
            acc = lax.fori_loop(0, K, k_body, init)
            for j in range(COLS):
                mx_v[n, pl.ds(j * 16, 16)] = acc[j]
                mn_v[n, pl.ds(j * 16, 16)] = acc[COLS + j]
        pltpu.sync_copy(mx_v, mx_hbm.at[pl.ds(nb, CN)])
        pltpu.sync_copy(mn_v, mn_hbm.at[pl.ds(nb, CN)])
        return carry

    lax.fori_loop(0, nchunk, chunk_body, 0)


def _sc_minmax(u, v, i1_flat, i0_flat, n_pad):
    npw = n_pad // NW
    nchunk = npw // CN
    mesh = plsc.VectorSubcoreMesh(core_axis_name="c", subcore_axis_name="s")
    kern = functools.partial(
        pl.kernel,
        out_type=(jax.ShapeDtypeStruct((n_pad, OUT), jnp.float32),
                  jax.ShapeDtypeStruct((n_pad, OUT), jnp.float32)),
        mesh=mesh,
        scratch_types=[
            pltpu.VMEM((CE,), jnp.int32),
            pltpu.VMEM((CE,), jnp.int32),
            pltpu.VMEM((CE, OUT), jnp.float32),
            pltpu.VMEM((CE, OUT), jnp.float32),
            pltpu.VMEM((CN, OUT), jnp.float32),
            pltpu.VMEM((CN, OUT), jnp.float32),
            pltpu.SemaphoreType.DMA,
            pltpu.SemaphoreType.DMA,
        ],
    )(functools.partial(_sc_body, nchunk, npw))
    return kern(u, v, i1_flat, i0_flat)


def _gelu(z):
    return 0.5 * z * (1.0 + lax.erf(z * 0.7071067811865476))


def _final_body(mx_ref, mn_ref, o_ref):
    o_ref[...] = jnp.maximum(_gelu(mx_ref[...]), _gelu(mn_ref[...]))


def _finalize(mx, mn, n_pad, bn):
    grid = (n_pad // bn,)
    return pl.pallas_call(
        _final_body,
        grid=grid,
        in_specs=[
            pl.BlockSpec((bn, OUT), lambda i: (i, 0)),
            pl.BlockSpec((bn, OUT), lambda i: (i, 0)),
        ],
        out_specs=pl.BlockSpec((bn, OUT), lambda i: (i, 0)),
        out_shape=jax.ShapeDtypeStruct((n_pad, OUT), jnp.float32),
    )(mx, mn)


def kernel(x, edge_index, W, bias):
    n = x.shape[2]
    xr = x.reshape(C, n)
    n_pad = -(-n // 1024) * 1024
    x_pad = jnp.pad(xr, ((0, 0), (0, n_pad - n)))
    idx = edge_index.reshape(2, n, K).astype(jnp.int32)
    pad_rows = ((0, n_pad - n), (0, 0))
    i1_flat = jnp.pad(idx[1], pad_rows).reshape(-1)
    i0_flat = jnp.pad(idx[0], pad_rows).reshape(-1)
    w1 = W[:, :C]
    w2 = W[:, C:]
    a1 = w1 - w2
    u, v = _make_tables(x_pad, a1, w2, bias.reshape(1, OUT), n_pad, 1024)
    u, v, i1_flat, i0_flat = lax.optimization_barrier((u, v, i1_flat, i0_flat))
    mx, mn = _sc_minmax(u, v, i1_flat, i0_flat, n_pad)
    mx, mn = lax.optimization_barrier((mx, mn))
    g = _finalize(mx, mn, n_pad, 1024)
    return g[:n].T.reshape(1, OUT, n, 1)


# speedup vs baseline: 3.1369x; 1.2008x over previous
import functools

import jax
import jax.numpy as jnp
from jax import lax
from jax.experimental import pallas as pl
from jax.experimental.pallas import tpu as pltpu
from jax.experimental.pallas import tpu_sc as plsc

C = 128
OUT = 128
K = 32
NW = 32
CN = 4
CE = CN * K
COLS = OUT // 16


def _tables_body(x_ref, a1_ref, a2_ref, b_ref, u_ref, v_ref):
    x = x_ref[...]
    u = lax.dot_general(x, a1_ref[...], (((0,), (1,)), ((), ())),
                        preferred_element_type=jnp.float32)
    v = lax.dot_general(x, a2_ref[...], (((0,), (1,)), ((), ())),
                        preferred_element_type=jnp.float32)
    u_ref[...] = u + b_ref[...]
    v_ref[...] = v


def _make_tables(x_pad, a1, a2, b, n_pad, bn):
    grid = (n_pad // bn,)
    return pl.pallas_call(
        _tables_body,
        grid=grid,
        in_specs=[
            pl.BlockSpec((C, bn), lambda i: (0, i)),
            pl.BlockSpec((OUT, C), lambda i: (0, 0)),
            pl.BlockSpec((OUT, C), lambda i: (0, 0)),
            pl.BlockSpec((1, OUT), lambda i: (0, 0)),
        ],
        out_specs=[
            pl.BlockSpec((bn, OUT), lambda i: (i, 0)),
            pl.BlockSpec((bn, OUT), lambda i: (i, 0)),
        ],
        out_shape=[
            jax.ShapeDtypeStruct((n_pad, OUT), jnp.float32),
            jax.ShapeDtypeStruct((n_pad, OUT), jnp.float32),
        ],
    )(x_pad, a1, a2, b)


def _sc_body(nchunk, npw,
             u_hbm, v_hbm, i1_hbm, i0_hbm, mx_hbm, mn_hbm,
             i1x, i0x, ur, vr, smx, smn,
             gu0, gu1, gv0, gv1, sidx, so0, so1):
    cid = lax.axis_index("c")
    sid = lax.axis_index("s")
    wid = sid * 2 + cid
    base = wid * npw
    cbase = wid * nchunk
    npair = nchunk // 2
    gu = (gu0, gu1)
    gv = (gv0, gv1)
    so = (so0, so1)

    def fetch_idx(p, par):
        c1 = pltpu.async_copy(i1_hbm.at[pl.ds(cbase + 2 * p, 2)], i1x.at[par], sidx)
        c2 = pltpu.async_copy(i0_hbm.at[pl.ds(cbase + 2 * p, 2)], i0x.at[par], sidx)
        return c1, c2

    def wait_idx(par):
        pltpu.make_async_copy(i1_hbm.at[pl.ds(cbase, 2)], i1x.at[par], sidx).wait()
        pltpu.make_async_copy(i0_hbm.at[pl.ds(cbase, 2)], i0x.at[par], sidx).wait()

    def issue_gather(par, b):
        pltpu.async_copy(u_hbm.at[i1x.at[par, b]], ur.at[b], gu[b])
        pltpu.async_copy(v_hbm.at[i0x.at[par, b]], vr.at[b], gv[b])

    def wait_gather(par, b):
        pltpu.make_async_copy(u_hbm.at[i1x.at[par, b]], ur.at[b], gu[b]).wait()
        pltpu.make_async_copy(v_hbm.at[i0x.at[par, b]], vr.at[b], gv[b]).wait()

    def flush_out(par, p):
        nb = base + p * (2 * CN)
        pltpu.async_copy(smx.at[par], mx_hbm.at[pl.ds(nb, 2 * CN)], so[par])
        pltpu.async_copy(smn.at[par], mn_hbm.at[pl.ds(nb, 2 * CN)], so[par])

    def wait_out(par):
        pltpu.make_async_copy(smx.at[par], mx_hbm.at[pl.ds(base, 2 * CN)], so[par]).wait()
        pltpu.make_async_copy(smn.at[par], mn_hbm.at[pl.ds(base, 2 * CN)], so[par]).wait()

    def compute(b, par):
        for n in range(CN):
            def k_body(k, acc):
                row = n * K + k
                out = []
                for j in range(COLS):
                    z = (ur[b, row, pl.ds(j * 16, 16)]
                         + vr[b, row, pl.ds(j * 16, 16)])
                    out.append(jnp.maximum(acc[j], z))
                    out.append(jnp.minimum(acc[COLS + j], z))
                return tuple(out[0::2]) + tuple(out[1::2])

            init = (tuple(jnp.full((16,), -jnp.inf, jnp.float32) for _ in range(COLS))
                    + tuple(jnp.full((16,), jnp.inf, jnp.float32) for _ in range(COLS)))
            acc = lax.fori_loop(0, K, k_body, init)
            for j in range(COLS):
                smx[par, b * CN + n, pl.ds(j * 16, 16)] = acc[j]
                smn[par, b * CN + n, pl.ds(j * 16, 16)] = acc[COLS + j]

    c1, c2 = fetch_idx(0, 0)
    c1.wait()
    c2.wait()
    issue_gather(0, 0)
    issue_gather(0, 1)
    c1, c2 = fetch_idx(1, 1)
    c1.wait()
    c2.wait()

    def quad(qi, carry):
        for pp in range(2):
            p = 2 * qi + pp

            @pl.when(jnp.logical_and(p >= 1, p + 1 < npair))
            def _():
                wait_idx(1 - pp)

            for b in range(2):
                wait_gather(pp, b)
                if b == 0:
                    @pl.when(p >= 2)
                    def _():
                        wait_out(pp)
                compute(b, pp)

                @pl.when(p + 1 < npair)
                def _():
                    issue_gather(1 - pp, b)

            flush_out(pp, p)

            @pl.when(p + 2 < npair)
            def _():
                fetch_idx(p + 2, pp)
        return carry

    lax.fori_loop(0, npair // 2, quad, 0)
    wait_out(0)
    wait_out(1)


def _sc_minmax(u, v, i1_rows, i0_rows, n_pad):
    npw = n_pad // NW
    nchunk = npw // CN
    mesh = plsc.VectorSubcoreMesh(core_axis_name="c", subcore_axis_name="s")
    kern = functools.partial(
        pl.kernel,
        out_type=(jax.ShapeDtypeStruct((n_pad, OUT), jnp.float32),
                  jax.ShapeDtypeStruct((n_pad, OUT), jnp.float32)),
        mesh=mesh,
        scratch_types=[
            pltpu.VMEM((2, 2, CE), jnp.int32),
            pltpu.VMEM((2, 2, CE), jnp.int32),
            pltpu.VMEM((2, CE, OUT), jnp.float32),
            pltpu.VMEM((2, CE, OUT), jnp.float32),
            pltpu.VMEM((2, 2 * CN, OUT), jnp.float32),
            pltpu.VMEM((2, 2 * CN, OUT), jnp.float32),
            pltpu.SemaphoreType.DMA,
            pltpu.SemaphoreType.DMA,
            pltpu.SemaphoreType.DMA,
            pltpu.SemaphoreType.DMA,
            pltpu.SemaphoreType.DMA,
            pltpu.SemaphoreType.DMA,
            pltpu.SemaphoreType.DMA,
        ],
    )(functools.partial(_sc_body, nchunk, npw))
    return kern(u, v, i1_rows, i0_rows)


def _gelu(z):
    return 0.5 * z * (1.0 + lax.erf(z * 0.7071067811865476))


def _final_body(mx_ref, mn_ref, o_ref):
    o_ref[...] = jnp.maximum(_gelu(mx_ref[...]), _gelu(mn_ref[...]))


def _finalize(mx, mn, n_pad, bn):
    grid = (n_pad // bn,)
    return pl.pallas_call(
        _final_body,
        grid=grid,
        in_specs=[
            pl.BlockSpec((bn, OUT), lambda i: (i, 0)),
            pl.BlockSpec((bn, OUT), lambda i: (i, 0)),
        ],
        out_specs=pl.BlockSpec((bn, OUT), lambda i: (i, 0)),
        out_shape=jax.ShapeDtypeStruct((n_pad, OUT), jnp.float32),
    )(mx, mn)


def kernel(x, edge_index, W, bias):
    n = x.shape[2]
    xr = x.reshape(C, n)
    n_pad = -(-n // 1024) * 1024
    x_pad = jnp.pad(xr, ((0, 0), (0, n_pad - n)))
    idx = edge_index.reshape(2, n, K).astype(jnp.int32)
    pad_rows = ((0, n_pad - n), (0, 0))
    i1_flat = jnp.pad(idx[1], pad_rows).reshape(-1, CE)
    i0_flat = jnp.pad(idx[0], pad_rows).reshape(-1, CE)
    w1 = W[:, :C]
    w2 = W[:, C:]
    a1 = w1 - w2
    u, v = _make_tables(x_pad, a1, w2, bias.reshape(1, OUT), n_pad, 1024)
    u, v, i1_flat, i0_flat = lax.optimization_barrier((u, v, i1_flat, i0_flat))
    mx, mn = _sc_minmax(u, v, i1_flat, i0_flat, n_pad)
    mx, mn = lax.optimization_barrier((mx, mn))
    g = _finalize(mx, mn, n_pad, 1024)
    return g[:n].T.reshape(1, OUT, n, 1)
